# Initial kernel scaffold; baseline (speedup 1.0000x reference)
import jax
import jax.numpy as jnp
from jax.experimental import pallas as pl


def kernel(x, edge_index, W1, al1, ar1, W2, al2, ar2):
    raise NotImplementedError("write your pallas kernel here")


# speedup vs baseline: 39.4933x; 39.4933x over previous
import functools

import jax
import jax.numpy as jnp
from jax import lax
from jax.experimental import pallas as pl
from jax.experimental.pallas import tpu as pltpu
from jax.experimental.pallas import tpu_sc as plsc

N = 10000
E = 320000
D = 128
H = 2
NP = 10240
BLK = 1024
NB = NP // BLK
EPS = 1e-9


def _proj1_body(x_ref, w_ref, al_ref, ar_ref, z_ref, el_ref, er_ref):
    xb = x_ref[...]
    for h in range(H):
        zb = jnp.dot(xb, w_ref[:, h * D:(h + 1) * D],
                     preferred_element_type=jnp.float32)
        z_ref[h] = zb
        el_ref[h, :] = jnp.sum(zb * al_ref[h, :][None, :], axis=1)
        er_ref[h, :] = jnp.sum(zb * ar_ref[h, :][None, :], axis=1)


_proj1 = pl.pallas_call(
    _proj1_body,
    grid=(NB,),
    in_specs=[
        pl.BlockSpec((BLK, D), lambda i: (i, 0)),
        pl.BlockSpec((D, H * D), lambda i: (0, 0)),
        pl.BlockSpec((H, D), lambda i: (0, 0)),
        pl.BlockSpec((H, D), lambda i: (0, 0)),
    ],
    out_specs=[
        pl.BlockSpec((H, BLK, D), lambda i: (0, i, 0)),
        pl.BlockSpec((H, BLK), lambda i: (0, i)),
        pl.BlockSpec((H, BLK), lambda i: (0, i)),
    ],
    out_shape=[
        jax.ShapeDtypeStruct((H, NP, D), jnp.float32),
        jax.ShapeDtypeStruct((H, NP), jnp.float32),
        jax.ShapeDtypeStruct((H, NP), jnp.float32),
    ],
)


def _elu(v):
    return jnp.where(v > 0, v, jnp.exp(jnp.minimum(v, 0.0)) - 1.0)


def _proj2_body(acc_ref, den_ref, w_ref, al_ref, ar_ref, z_ref, el_ref, er_ref):
    d0 = den_ref[0, :][:, None] + EPS
    d1 = den_ref[1, :][:, None] + EPS
    h0 = _elu(acc_ref[0] / d0)
    h1 = _elu(acc_ref[1] / d1)
    zb = jnp.dot(h0, w_ref[:D, :], preferred_element_type=jnp.float32)
    zb = zb + jnp.dot(h1, w_ref[D:, :], preferred_element_type=jnp.float32)
    z_ref[...] = zb
    el_ref[0, :] = jnp.sum(zb * al_ref[0, :][None, :], axis=1)
    er_ref[0, :] = jnp.sum(zb * ar_ref[0, :][None, :], axis=1)


_proj2 = pl.pallas_call(
    _proj2_body,
    grid=(NB,),
    in_specs=[
        pl.BlockSpec((2, BLK, D), lambda i: (0, i, 0)),
        pl.BlockSpec((2, BLK), lambda i: (0, i)),
        pl.BlockSpec((H * D, D), lambda i: (0, 0)),
        pl.BlockSpec((1, D), lambda i: (0, 0)),
        pl.BlockSpec((1, D), lambda i: (0, 0)),
    ],
    out_specs=[
        pl.BlockSpec((BLK, D), lambda i: (i, 0)),
        pl.BlockSpec((1, BLK), lambda i: (0, i)),
        pl.BlockSpec((1, BLK), lambda i: (0, i)),
    ],
    out_shape=[
        jax.ShapeDtypeStruct((NP, D), jnp.float32),
        jax.ShapeDtypeStruct((1, NP), jnp.float32),
        jax.ShapeDtypeStruct((1, NP), jnp.float32),
    ],
)


def _merge_body(acc_ref, den_ref, o_ref):
    dsum = (den_ref[0, :] + den_ref[1, :])[:, None] + EPS
    o_ref[...] = (acc_ref[0] + acc_ref[1]) / dsum


_merge = pl.pallas_call(
    _merge_body,
    grid=(NB,),
    in_specs=[
        pl.BlockSpec((2, BLK, D), lambda i: (0, i, 0)),
        pl.BlockSpec((2, BLK), lambda i: (0, i)),
    ],
    out_specs=pl.BlockSpec((BLK, D), lambda i: (i, 0)),
    out_shape=jax.ShapeDtypeStruct((NP, D), jnp.float32),
)


B = 256
CH = 128
NSUB = 16
ZCH = 320
ZPT = NP // NSUB


def _make_sc_edge(ec, gstride, sstride):
    tb = ec // B

    mesh = plsc.VectorSubcoreMesh(core_axis_name="c", subcore_axis_name="s")

    @functools.partial(
        pl.kernel,
        out_type=[
            jax.ShapeDtypeStruct((2, NP, D), jnp.float32),
            jax.ShapeDtypeStruct((2, NP), jnp.float32),
        ],
        mesh=mesh,
        scratch_types=[
            pltpu.VMEM((B // CH, 1, CH), jnp.int32),
            pltpu.VMEM((B // CH, 1, CH), jnp.int32),
            pltpu.VMEM((B // CH, 1, CH), jnp.int32),
            pltpu.VMEM((B,), jnp.float32),
            pltpu.VMEM((B,), jnp.float32),
            pltpu.VMEM((B,), jnp.float32),
            pltpu.VMEM((B, D), jnp.float32),
            pltpu.VMEM_SHARED((NP, D), jnp.float32),
            pltpu.VMEM_SHARED((NP,), jnp.float32),
            pltpu.SemaphoreType.DMA,
            pltpu.SemaphoreType.DMA,
        ],
    )
    def k(srcg, dstg, dstph, elf, erf, ztab, acc_out, den_out,
          srcv, dstv, dstp, elg, erg, ee, rows, acc_sh, den_sh, sA, sB):
        c = lax.axis_index("c")
        s = lax.axis_index("s")

        z16 = jnp.zeros((16,), jnp.float32)

        def zrow(i, carry):
            for j in range(D // 16):
                rows[i, pl.ds(j * 16, 16)] = z16
            return carry

        lax.fori_loop(0, ZCH, zrow, 0)

        def zee(i, carry):
            ee[pl.ds(i * 16, 16)] = z16
            return carry

        lax.fori_loop(0, ZCH // 16, zee, 0)

        for t in range(ZPT // ZCH):
            pltpu.sync_copy(rows.at[pl.ds(0, ZCH)],
                            acc_sh.at[pl.ds(s * ZPT + t * ZCH, ZCH)])
            pltpu.sync_copy(ee.at[pl.ds(0, ZCH)],
                            den_sh.at[pl.ds(s * ZPT + t * ZCH, ZCH)])
        plsc.subcore_barrier()

        grow0 = (c * gstride) // CH
        srow0 = (c * sstride) // CH
        rpb = B // CH

        def eblk(bi, carry):
            t = bi * NSUB + s
            pltpu.sync_copy(srcg.at[pl.ds(grow0 + t * rpb, rpb)], srcv)
            pltpu.sync_copy(dstg.at[pl.ds(grow0 + t * rpb, rpb)], dstv)
            pltpu.sync_copy(dstph.at[pl.ds(srow0 + t * rpb, rpb)], dstp)
            cps = []
            for j in range(rpb):
                cps.append(pltpu.async_copy(
                    elf.at[srcv.at[j, 0]], elg.at[pl.ds(j * CH, CH)], sA))
                cps.append(pltpu.async_copy(
                    erf.at[dstv.at[j, 0]], erg.at[pl.ds(j * CH, CH)], sA))
                cps.append(pltpu.async_copy(
                    ztab.at[srcv.at[j, 0]], rows.at[pl.ds(j * CH, CH)], sB))
            for j in range(rpb):
                cps[3 * j].wait()
                cps[3 * j + 1].wait()
            for g in range(B // 16):
                sl = pl.ds(g * 16, 16)
                sv = elg[sl] + erg[sl]
                ev = jnp.maximum(sv, 0.2 * sv)
                ee[sl] = jnp.exp(ev)
            for j in range(rpb):
                pltpu.sync_copy(ee.at[pl.ds(j * CH, CH)],
                                den_sh.at[dstp.at[j, 0]], add=True)
            for j in range(rpb):
                cps[3 * j + 2].wait()

            def scale(g, cy):
                ev = ee[pl.ds(g * 16, 16)]
                for i in range(16):
                    a = ev[i]
                    for j in range(D // 16):
                        sl2 = pl.ds(j * 16, 16)
                        rows[g * 16 + i, sl2] = rows[g * 16 + i, sl2] * a
                return cy

            lax.fori_loop(0, B // 16, scale, 0)
            for j in range(rpb):
                pltpu.sync_copy(rows.at[pl.ds(j * CH, CH)],
                                acc_sh.at[dstp.at[j, 0]], add=True)
            return carry

        nblk = (tb - s + NSUB - 1) // NSUB
        lax.fori_loop(0, nblk, eblk, 0)
        plsc.subcore_barrier()

        pltpu.sync_copy(acc_sh.at[pl.ds(s * ZPT, ZPT)],
                        acc_out.at[c, pl.ds(s * ZPT, ZPT)])
        pltpu.sync_copy(den_sh.at[pl.ds(s * ZPT, ZPT)],
                        den_out.at[c, pl.ds(s * ZPT, ZPT)])

    return k


_sc_layer1 = _make_sc_edge(E, E, 0)
_sc_layer2 = _make_sc_edge(E // 2, E // 2, E // 2)


def kernel(x, edge_index, W1, al1, ar1, W2, al2, ar2):
    src = edge_index[0].astype(jnp.int32)
    dst = edge_index[1].astype(jnp.int32)
    srcadj = jnp.concatenate([src, src + NP]).reshape(-1, 1, CH)
    dstadj = jnp.concatenate([dst, dst + NP]).reshape(-1, 1, CH)
    src2 = src.reshape(-1, 1, CH)
    dst2 = dst.reshape(-1, 1, CH)

    xp = jnp.pad(x, ((0, NP - N), (0, 0)))
    z1, el1, er1 = _proj1(xp, W1, al1, ar1)
    acc1, den1 = _sc_layer1(srcadj, dstadj, dst2,
                            el1.reshape(-1), er1.reshape(-1),
                            z1.reshape(H * NP, D))
    z2, el2, er2 = _proj2(acc1, den1, W2, al2, ar2)
    acc2, den2 = _sc_layer2(src2, dst2, dst2,
                            el2.reshape(-1), er2.reshape(-1), z2)
    out = _merge(acc2, den2)
    return out[:N]


# speedup vs baseline: 53.2647x; 1.3487x over previous
import functools

import jax
import jax.numpy as jnp
from jax import lax
from jax.experimental import pallas as pl
from jax.experimental.pallas import tpu as pltpu
from jax.experimental.pallas import tpu_sc as plsc

N = 10000
E = 320000
D = 128
H = 2
NP = 10240
BLK = 1024
NB = NP // BLK
EPS = 1e-9


def _proj1_body(x_ref, w_ref, al_ref, ar_ref, z_ref, el_ref, er_ref):
    xb = x_ref[...]
    for h in range(H):
        zb = jnp.dot(xb, w_ref[:, h * D:(h + 1) * D],
                     preferred_element_type=jnp.float32)
        z_ref[h] = zb
        el_ref[h, :] = jnp.sum(zb * al_ref[h, :][None, :], axis=1)
        er_ref[h, :] = jnp.sum(zb * ar_ref[h, :][None, :], axis=1)


_proj1 = pl.pallas_call(
    _proj1_body,
    grid=(NB,),
    in_specs=[
        pl.BlockSpec((BLK, D), lambda i: (i, 0)),
        pl.BlockSpec((D, H * D), lambda i: (0, 0)),
        pl.BlockSpec((H, D), lambda i: (0, 0)),
        pl.BlockSpec((H, D), lambda i: (0, 0)),
    ],
    out_specs=[
        pl.BlockSpec((H, BLK, D), lambda i: (0, i, 0)),
        pl.BlockSpec((H, BLK), lambda i: (0, i)),
        pl.BlockSpec((H, BLK), lambda i: (0, i)),
    ],
    out_shape=[
        jax.ShapeDtypeStruct((H, NP, D), jnp.float32),
        jax.ShapeDtypeStruct((H, NP), jnp.float32),
        jax.ShapeDtypeStruct((H, NP), jnp.float32),
    ],
)


def _elu(v):
    return jnp.where(v > 0, v, jnp.exp(jnp.minimum(v, 0.0)) - 1.0)


def _proj2_body(acc_ref, den_ref, w_ref, al_ref, ar_ref, z_ref, el_ref, er_ref):
    d0 = den_ref[0, :][:, None] + EPS
    d1 = den_ref[1, :][:, None] + EPS
    h0 = _elu(acc_ref[0] / d0)
    h1 = _elu(acc_ref[1] / d1)
    zb = jnp.dot(h0, w_ref[:D, :], preferred_element_type=jnp.float32)
    zb = zb + jnp.dot(h1, w_ref[D:, :], preferred_element_type=jnp.float32)
    z_ref[...] = zb
    el_ref[0, :] = jnp.sum(zb * al_ref[0, :][None, :], axis=1)
    er_ref[0, :] = jnp.sum(zb * ar_ref[0, :][None, :], axis=1)


_proj2 = pl.pallas_call(
    _proj2_body,
    grid=(NB,),
    in_specs=[
        pl.BlockSpec((2, BLK, D), lambda i: (0, i, 0)),
        pl.BlockSpec((2, BLK), lambda i: (0, i)),
        pl.BlockSpec((H * D, D), lambda i: (0, 0)),
        pl.BlockSpec((1, D), lambda i: (0, 0)),
        pl.BlockSpec((1, D), lambda i: (0, 0)),
    ],
    out_specs=[
        pl.BlockSpec((BLK, D), lambda i: (i, 0)),
        pl.BlockSpec((1, BLK), lambda i: (0, i)),
        pl.BlockSpec((1, BLK), lambda i: (0, i)),
    ],
    out_shape=[
        jax.ShapeDtypeStruct((NP, D), jnp.float32),
        jax.ShapeDtypeStruct((1, NP), jnp.float32),
        jax.ShapeDtypeStruct((1, NP), jnp.float32),
    ],
)


def _merge_body(acc_ref, den_ref, o_ref):
    dsum = (den_ref[0, :] + den_ref[1, :])[:, None] + EPS
    o_ref[...] = (acc_ref[0] + acc_ref[1]) / dsum


_merge = pl.pallas_call(
    _merge_body,
    grid=(NB,),
    in_specs=[
        pl.BlockSpec((2, BLK, D), lambda i: (0, i, 0)),
        pl.BlockSpec((2, BLK), lambda i: (0, i)),
    ],
    out_specs=pl.BlockSpec((BLK, D), lambda i: (i, 0)),
    out_shape=jax.ShapeDtypeStruct((NP, D), jnp.float32),
)


B = 128
CH = 128
NSUB = 16
ZCH = 128
ZPT = NP // NSUB


def _make_sc_edge(ec):
    tb = ec // B
    maxblk = (tb + NSUB - 1) // NSUB

    mesh = plsc.VectorSubcoreMesh(core_axis_name="c", subcore_axis_name="s")

    @functools.partial(
        pl.kernel,
        out_type=[
            jax.ShapeDtypeStruct((2, NP, D), jnp.float32),
            jax.ShapeDtypeStruct((2, NP), jnp.float32),
        ],
        mesh=mesh,
        scratch_types=[
            pltpu.VMEM((3, 1, CH), jnp.int32),
            pltpu.VMEM((3, 1, CH), jnp.int32),
            pltpu.VMEM((B,), jnp.float32),
            pltpu.VMEM((B,), jnp.float32),
            pltpu.VMEM((B,), jnp.float32),
            pltpu.VMEM((B,), jnp.float32),
            pltpu.VMEM((B,), jnp.float32),
            pltpu.VMEM((B, D), jnp.float32),
            pltpu.VMEM((B, D), jnp.float32),
            pltpu.VMEM_SHARED((NP, D), jnp.float32),
            pltpu.VMEM_SHARED((NP,), jnp.float32),
            pltpu.SemaphoreType.DMA,
            pltpu.SemaphoreType.DMA,
            pltpu.SemaphoreType.DMA,
            pltpu.SemaphoreType.DMA,
            pltpu.SemaphoreType.DMA,
            pltpu.SemaphoreType.DMA,
        ],
    )
    def k(idxall, elf, erf, ztab, acc_out, den_out,
          idx0, idx1, elg0, elg1, erg0, erg1, ee, rows0, rows1,
          acc_sh, den_sh, sA0, sA1, sB0, sB1, sC0, sC1):
        idx = (idx0, idx1)
        elg = (elg0, elg1)
        erg = (erg0, erg1)
        rows = (rows0, rows1)
        semA = (sA0, sA1)
        semB = (sB0, sB1)
        semC = (sC0, sC1)
        c = lax.axis_index("c")
        s = lax.axis_index("s")

        z16 = jnp.zeros((16,), jnp.float32)

        def zrow(i, carry):
            for j in range(D // 16):
                rows0[i, pl.ds(j * 16, 16)] = z16
            return carry

        lax.fori_loop(0, ZCH, zrow, 0)

        def zee(i, carry):
            elg0[pl.ds(i * 16, 16)] = z16
            return carry

        lax.fori_loop(0, B // 16, zee, 0)

        for t in range(ZPT // ZCH):
            pltpu.sync_copy(rows0, acc_sh.at[pl.ds(s * ZPT + t * ZCH, ZCH)])
            pltpu.sync_copy(elg0, den_sh.at[pl.ds(s * ZPT + t * ZCH, B)])
        plsc.subcore_barrier()

        rbase = c * tb * 3
        nblk = (tb - s + NSUB - 1) // NSUB

        def load_idx(b, bi):
            t = bi * NSUB + s
            pltpu.sync_copy(idxall.at[pl.ds(rbase + t * 3, 3)], idx[b])

        def start_gathers(b):
            pltpu.async_copy(elf.at[idx[b].at[0, 0]], elg[b], semA[b])
            pltpu.async_copy(erf.at[idx[b].at[1, 0]], erg[b], semA[b])
            pltpu.async_copy(ztab.at[idx[b].at[0, 0]], rows[b], semB[b])

        def drain_scatter(b):
            pltpu.make_async_copy(rows[b], acc_sh.at[idx[b].at[2, 0]],
                                  semC[b]).wait()

        def consume(b):
            pltpu.make_async_copy(elf.at[idx[b].at[0, 0]], elg[b],
                                  semA[b]).wait()
            pltpu.make_async_copy(erf.at[idx[b].at[1, 0]], erg[b],
                                  semA[b]).wait()
            for g in range(B // 16):
                sl = pl.ds(g * 16, 16)
                sv = elg[b][sl] + erg[b][sl]
                ev = jnp.maximum(sv, 0.2 * sv)
                ee[sl] = jnp.exp(ev)
            pltpu.sync_copy(ee, den_sh.at[idx[b].at[2, 0]], add=True)
            pltpu.make_async_copy(ztab.at[idx[b].at[0, 0]], rows[b],
                                  semB[b]).wait()

            def scale(g, cy):
                ev = ee[pl.ds(g * 16, 16)]
                for i in range(16):
                    a = ev[i]
                    for j in range(D // 16):
                        sl2 = pl.ds(j * 16, 16)
                        rows[b][g * 16 + i, sl2] = rows[b][g * 16 + i, sl2] * a
                return cy

            lax.fori_loop(0, B // 16, scale, 0)
            pltpu.async_copy(rows[b], acc_sh.at[idx[b].at[2, 0]], semC[b],
                             add=True)

        @pl.when(0 < nblk)
        def _():
            load_idx(0, 0)
            start_gathers(0)

        def slot(r, bi):
            o = 1 - r

            @pl.when(bi < nblk)
            def _():
                @pl.when(bi >= 1)
                def _():
                    drain_scatter(o)

                @pl.when(bi + 1 < nblk)
                def _():
                    load_idx(o, bi + 1)
                    start_gathers(o)
                consume(r)

        def ring(g2, carry):
            slot(0, g2 * 2)
            slot(1, g2 * 2 + 1)
            return carry

        lax.fori_loop(0, (maxblk + 1) // 2, ring, 0)

        for r in range(2):
            @pl.when((nblk >= 1) & ((nblk - 1) % 2 == r))
            def _(r=r):
                drain_scatter(r)
        plsc.subcore_barrier()

        pltpu.sync_copy(acc_sh.at[pl.ds(s * ZPT, ZPT)],
                        acc_out.at[c, pl.ds(s * ZPT, ZPT)])
        pltpu.sync_copy(den_sh.at[pl.ds(s * ZPT, ZPT)],
                        den_out.at[c, pl.ds(s * ZPT, ZPT)])

    return k


_sc_layer1 = _make_sc_edge(E)
_sc_layer2 = _make_sc_edge(E // 2)


def kernel(x, edge_index, W1, al1, ar1, W2, al2, ar2):
    src = edge_index[0].astype(jnp.int32)
    dst = edge_index[1].astype(jnp.int32)
    srcr = src.reshape(-1, 1, CH)
    dstr = dst.reshape(-1, 1, CH)
    idx1 = jnp.stack(
        [jnp.concatenate([srcr, srcr + NP]),
         jnp.concatenate([dstr, dstr + NP]),
         jnp.concatenate([dstr, dstr])], axis=1).reshape(-1, 1, CH)
    idx2 = jnp.stack([srcr, dstr, dstr], axis=1).reshape(-1, 1, CH)

    xp = jnp.pad(x, ((0, NP - N), (0, 0)))
    z1, el1, er1 = _proj1(xp, W1, al1, ar1)
    acc1, den1 = _sc_layer1(idx1, el1.reshape(-1), er1.reshape(-1),
                            z1.reshape(H * NP, D))
    z2, el2, er2 = _proj2(acc1, den1, W2, al2, ar2)
    acc2, den2 = _sc_layer2(idx2, el2.reshape(-1), er2.reshape(-1), z2)
    out = _merge(acc2, den2)
    return out[:N]


# speedup vs baseline: 54.9409x; 1.0315x over previous
import functools

import jax
import jax.numpy as jnp
from jax import lax
from jax.experimental import pallas as pl
from jax.experimental.pallas import tpu as pltpu
from jax.experimental.pallas import tpu_sc as plsc

N = 10000
E = 320000
D = 128
H = 2
NP = 10240
BLK = 1024
NB = NP // BLK
EPS = 1e-9


def _proj1_body(x_ref, w_ref, al_ref, ar_ref, z_ref, el_ref, er_ref):
    xb = x_ref[...]
    for h in range(H):
        zb = jnp.dot(xb, w_ref[:, h * D:(h + 1) * D],
                     preferred_element_type=jnp.float32)
        z_ref[h] = zb
        el_ref[h, :] = jnp.sum(zb * al_ref[h, :][None, :], axis=1)
        er_ref[h, :] = jnp.sum(zb * ar_ref[h, :][None, :], axis=1)


_proj1 = pl.pallas_call(
    _proj1_body,
    grid=(NB,),
    in_specs=[
        pl.BlockSpec((BLK, D), lambda i: (i, 0)),
        pl.BlockSpec((D, H * D), lambda i: (0, 0)),
        pl.BlockSpec((H, D), lambda i: (0, 0)),
        pl.BlockSpec((H, D), lambda i: (0, 0)),
    ],
    out_specs=[
        pl.BlockSpec((H, BLK, D), lambda i: (0, i, 0)),
        pl.BlockSpec((H, BLK), lambda i: (0, i)),
        pl.BlockSpec((H, BLK), lambda i: (0, i)),
    ],
    out_shape=[
        jax.ShapeDtypeStruct((H, NP, D), jnp.float32),
        jax.ShapeDtypeStruct((H, NP), jnp.float32),
        jax.ShapeDtypeStruct((H, NP), jnp.float32),
    ],
)


def _elu(v):
    return jnp.where(v > 0, v, jnp.exp(jnp.minimum(v, 0.0)) - 1.0)


def _proj2_body(acc_ref, den_ref, w_ref, al_ref, ar_ref, z_ref, el_ref, er_ref):
    d0 = den_ref[0, :][:, None] + EPS
    d1 = den_ref[1, :][:, None] + EPS
    h0 = _elu(acc_ref[0] / d0)
    h1 = _elu(acc_ref[1] / d1)
    zb = jnp.dot(h0, w_ref[:D, :], preferred_element_type=jnp.float32)
    zb = zb + jnp.dot(h1, w_ref[D:, :], preferred_element_type=jnp.float32)
    z_ref[...] = zb
    el_ref[0, :] = jnp.sum(zb * al_ref[0, :][None, :], axis=1)
    er_ref[0, :] = jnp.sum(zb * ar_ref[0, :][None, :], axis=1)


_proj2 = pl.pallas_call(
    _proj2_body,
    grid=(NB,),
    in_specs=[
        pl.BlockSpec((2, BLK, D), lambda i: (0, i, 0)),
        pl.BlockSpec((2, BLK), lambda i: (0, i)),
        pl.BlockSpec((H * D, D), lambda i: (0, 0)),
        pl.BlockSpec((1, D), lambda i: (0, 0)),
        pl.BlockSpec((1, D), lambda i: (0, 0)),
    ],
    out_specs=[
        pl.BlockSpec((BLK, D), lambda i: (i, 0)),
        pl.BlockSpec((1, BLK), lambda i: (0, i)),
        pl.BlockSpec((1, BLK), lambda i: (0, i)),
    ],
    out_shape=[
        jax.ShapeDtypeStruct((NP, D), jnp.float32),
        jax.ShapeDtypeStruct((1, NP), jnp.float32),
        jax.ShapeDtypeStruct((1, NP), jnp.float32),
    ],
)


def _merge_body(acc_ref, den_ref, o_ref):
    dsum = (den_ref[0, :] + den_ref[1, :])[:, None] + EPS
    o_ref[...] = (acc_ref[0] + acc_ref[1]) / dsum


_merge = pl.pallas_call(
    _merge_body,
    grid=(NB,),
    in_specs=[
        pl.BlockSpec((2, BLK, D), lambda i: (0, i, 0)),
        pl.BlockSpec((2, BLK), lambda i: (0, i)),
    ],
    out_specs=pl.BlockSpec((BLK, D), lambda i: (i, 0)),
    out_shape=jax.ShapeDtypeStruct((NP, D), jnp.float32),
)


B = 128
CH = 128
NSUB = 16
ZCH = 128
ZPT = NP // NSUB


def _make_sc_edge(ec):
    tb = ec // B
    maxblk = (tb + NSUB - 1) // NSUB

    mesh = plsc.VectorSubcoreMesh(core_axis_name="c", subcore_axis_name="s")

    @functools.partial(
        pl.kernel,
        out_type=[
            jax.ShapeDtypeStruct((2, NP, D), jnp.float32),
            jax.ShapeDtypeStruct((2, NP), jnp.float32),
        ],
        mesh=mesh,
        scratch_types=[
            pltpu.VMEM((3, 1, CH), jnp.int32),
            pltpu.VMEM((3, 1, CH), jnp.int32),
            pltpu.VMEM((B,), jnp.float32),
            pltpu.VMEM((B,), jnp.float32),
            pltpu.VMEM((B,), jnp.float32),
            pltpu.VMEM((B,), jnp.float32),
            pltpu.VMEM((B,), jnp.float32),
            pltpu.VMEM((B,), jnp.float32),
            pltpu.VMEM((B, D), jnp.float32),
            pltpu.VMEM((B, D), jnp.float32),
            pltpu.VMEM_SHARED((NP, D), jnp.float32),
            pltpu.VMEM_SHARED((NP,), jnp.float32),
            pltpu.SemaphoreType.DMA,
            pltpu.SemaphoreType.DMA,
            pltpu.SemaphoreType.DMA,
            pltpu.SemaphoreType.DMA,
            pltpu.SemaphoreType.DMA,
            pltpu.SemaphoreType.DMA,
        ],
    )
    def k(idxall, elf, erf, ztab, acc_out, den_out,
          idx0, idx1, elg0, elg1, erg0, erg1, ee0, ee1, rows0, rows1,
          acc_sh, den_sh, sA0, sA1, sB0, sB1, sC0, sC1):
        idx = (idx0, idx1)
        elg = (elg0, elg1)
        erg = (erg0, erg1)
        ee = (ee0, ee1)
        rows = (rows0, rows1)
        semA = (sA0, sA1)
        semB = (sB0, sB1)
        semC = (sC0, sC1)
        c = lax.axis_index("c")
        s = lax.axis_index("s")

        z16 = jnp.zeros((16,), jnp.float32)

        def zrow(i, carry):
            for j in range(D // 16):
                rows0[i, pl.ds(j * 16, 16)] = z16
            return carry

        lax.fori_loop(0, ZCH, zrow, 0)

        def zee(i, carry):
            elg0[pl.ds(i * 16, 16)] = z16
            return carry

        lax.fori_loop(0, B // 16, zee, 0)

        for t in range(ZPT // ZCH):
            pltpu.sync_copy(rows0, acc_sh.at[pl.ds(s * ZPT + t * ZCH, ZCH)])
            pltpu.sync_copy(elg0, den_sh.at[pl.ds(s * ZPT + t * ZCH, B)])
        plsc.subcore_barrier()

        rbase = c * tb * 3
        nblk = (tb - s + NSUB - 1) // NSUB

        def load_idx(b, bi):
            t = bi * NSUB + s
            pltpu.sync_copy(idxall.at[pl.ds(rbase + t * 3, 3)], idx[b])

        def start_gathers(b):
            pltpu.async_copy(elf.at[idx[b].at[0, 0]], elg[b], semA[b])
            pltpu.async_copy(erf.at[idx[b].at[1, 0]], erg[b], semA[b])
            pltpu.async_copy(ztab.at[idx[b].at[0, 0]], rows[b], semB[b])

        def drain_scatter(b):
            pltpu.make_async_copy(ee[b], den_sh.at[idx[b].at[2, 0]],
                                  semC[b]).wait()
            pltpu.make_async_copy(rows[b], acc_sh.at[idx[b].at[2, 0]],
                                  semC[b]).wait()

        def weights(b):
            pltpu.make_async_copy(elf.at[idx[b].at[0, 0]], elg[b],
                                  semA[b]).wait()
            pltpu.make_async_copy(erf.at[idx[b].at[1, 0]], erg[b],
                                  semA[b]).wait()
            for g in range(B // 16):
                sl = pl.ds(g * 16, 16)
                sv = elg[b][sl] + erg[b][sl]
                ev = jnp.maximum(sv, 0.2 * sv)
                ee[b][sl] = jnp.exp(ev)
            pltpu.async_copy(ee[b], den_sh.at[idx[b].at[2, 0]], semC[b],
                             add=True)

        def messages(b):
            pltpu.make_async_copy(ztab.at[idx[b].at[0, 0]], rows[b],
                                  semB[b]).wait()

            def scale(g, cy):
                ev = ee[b][pl.ds(g * 16, 16)]
                for i in range(16):
                    a = ev[i]
                    for j in range(D // 16):
                        sl2 = pl.ds(j * 16, 16)
                        rows[b][g * 16 + i, sl2] = rows[b][g * 16 + i, sl2] * a
                return cy

            lax.fori_loop(0, B // 16, scale, 0)
            pltpu.async_copy(rows[b], acc_sh.at[idx[b].at[2, 0]], semC[b],
                             add=True)

        @pl.when(0 < nblk)
        def _():
            load_idx(0, 0)
            start_gathers(0)

        def slot(r, bi):
            o = 1 - r

            @pl.when(bi < nblk)
            def _():
                weights(r)

                @pl.when(bi >= 1)
                def _():
                    drain_scatter(o)

                @pl.when(bi + 1 < nblk)
                def _():
                    load_idx(o, bi + 1)
                    start_gathers(o)
                messages(r)

        def ring(g2, carry):
            slot(0, g2 * 2)
            slot(1, g2 * 2 + 1)
            return carry

        lax.fori_loop(0, (maxblk + 1) // 2, ring, 0)

        for r in range(2):
            @pl.when((nblk >= 1) & ((nblk - 1) % 2 == r))
            def _(r=r):
                drain_scatter(r)
        plsc.subcore_barrier()

        pltpu.sync_copy(acc_sh.at[pl.ds(s * ZPT, ZPT)],
                        acc_out.at[c, pl.ds(s * ZPT, ZPT)])
        pltpu.sync_copy(den_sh.at[pl.ds(s * ZPT, ZPT)],
                        den_out.at[c, pl.ds(s * ZPT, ZPT)])

    return k


_sc_layer1 = _make_sc_edge(E)
_sc_layer2 = _make_sc_edge(E // 2)


def kernel(x, edge_index, W1, al1, ar1, W2, al2, ar2):
    src = edge_index[0].astype(jnp.int32)
    dst = edge_index[1].astype(jnp.int32)
    srcr = src.reshape(-1, 1, CH)
    dstr = dst.reshape(-1, 1, CH)
    idx1 = jnp.stack(
        [jnp.concatenate([srcr, srcr + NP]),
         jnp.concatenate([dstr, dstr + NP]),
         jnp.concatenate([dstr, dstr])], axis=1).reshape(-1, 1, CH)
    idx2 = jnp.stack([srcr, dstr, dstr], axis=1).reshape(-1, 1, CH)

    xp = jnp.pad(x, ((0, NP - N), (0, 0)))
    z1, el1, er1 = _proj1(xp, W1, al1, ar1)
    acc1, den1 = _sc_layer1(idx1, el1.reshape(-1), er1.reshape(-1),
                            z1.reshape(H * NP, D))
    z2, el2, er2 = _proj2(acc1, den1, W2, al2, ar2)
    acc2, den2 = _sc_layer2(idx2, el2.reshape(-1), er2.reshape(-1), z2)
    out = _merge(acc2, den2)
    return out[:N]
